# baseline (speedup 1.0000x reference)
import functools

import jax
import jax.numpy as jnp
from jax import lax
from jax.experimental import pallas as pl
from jax.experimental.pallas import tpu as pltpu
from jax.experimental.pallas import tpu_sc as plsc

NC = 2
NS = 16
NW = NC * NS
CHUNK = 125
PHASES = 5
RBLK = 80


def _sc_segment_sum(x, edges3d):
    n, f = x.shape
    _, chunks_total, chunk = edges3d.shape
    chunks_w = chunks_total // NW
    nblk = n // RBLK
    blk_iters = (nblk + NS - 1) // NS
    phases = PHASES
    chunks_ph = chunks_w // phases
    mesh = plsc.VectorSubcoreMesh(core_axis_name="c", subcore_axis_name="s")

    @functools.partial(
        pl.kernel,
        out_type=jax.ShapeDtypeStruct((NC, n, f), jnp.float32),
        mesh=mesh,
        scratch_types=[
            pltpu.VMEM((chunks_ph, chunk), jnp.int32),
            pltpu.VMEM((chunks_ph, chunk), jnp.int32),
            pltpu.VMEM((chunks_ph, chunk), jnp.int32),
            pltpu.VMEM((chunks_ph, chunk), jnp.int32),
            pltpu.VMEM((chunk, f), jnp.float32),
            pltpu.VMEM((chunk, f), jnp.float32),
            pltpu.VMEM_SHARED((n, f), jnp.float32),
            pltpu.SemaphoreType.DMA,
            pltpu.SemaphoreType.DMA,
            pltpu.SemaphoreType.DMA,
        ],
    )
    def seg_sum(x_hbm, e_hbm, out_hbm, src_a, dst_a, src_b, dst_b, rows0, rows1,
                acc, sem0, sem1, zsem):
        c = lax.axis_index("c")
        s = lax.axis_index("s")
        wid = c * NS + s

        zeros16 = jnp.zeros((16,), jnp.float32)

        def zero_body(i, carry):
            r = i // (f // 16)
            l = i - r * (f // 16)
            rows0[r, pl.ds(l * 16, 16)] = zeros16
            return carry

        lax.fori_loop(0, RBLK * (f // 16), zero_body, 0)

        def zero_acc_body(j, carry):
            blk = s + j * NS

            @pl.when(blk < nblk)
            def _():
                pltpu.async_copy(rows0.at[pl.ds(0, RBLK)],
                                 acc.at[pl.ds(blk * RBLK, RBLK)], zsem)

            return carry

        lax.fori_loop(0, blk_iters, zero_acc_body, 0)

        pltpu.sync_copy(e_hbm.at[0, pl.ds(wid * chunks_w, chunks_ph)], src_a)
        pltpu.sync_copy(e_hbm.at[1, pl.ds(wid * chunks_w, chunks_ph)], dst_a)
        pltpu.async_copy(x_hbm.at[src_a.at[0]], rows1, sem1)

        def zero_drain_body(j, carry):
            blk = s + j * NS

            @pl.when(blk < nblk)
            def _():
                pltpu.make_async_copy(rows0.at[pl.ds(0, RBLK)],
                                      acc.at[pl.ds(blk * RBLK, RBLK)], zsem).wait()

            return carry

        lax.fori_loop(0, blk_iters, zero_drain_body, 0)

        plsc.subcore_barrier()

        n_pairs = chunks_ph // 2
        bufs = ((rows1, sem1), (rows0, sem0))
        pltpu.async_copy(x_hbm.at[src_a.at[1]], rows0, sem0)
        idx_sets = [(src_a, dst_a), (src_b, dst_b)]
        for phase in range(phases):
            cur_src, cur_dst = idx_sets[phase % 2]
            nxt_src, nxt_dst = idx_sets[(phase + 1) % 2]
            last = phase == phases - 1
            if not last:
                base = wid * chunks_w + (phase + 1) * chunks_ph
                pltpu.sync_copy(e_hbm.at[0, pl.ds(base, chunks_ph)], nxt_src)
                pltpu.sync_copy(e_hbm.at[1, pl.ds(base, chunks_ph)], nxt_dst)

            def pair_body(i, carry, cur_src=cur_src, cur_dst=cur_dst,
                          nxt_src=nxt_src, last=last):
                for b, (buf, sem) in enumerate(bufs):
                    j = 2 * i + b
                    pltpu.make_async_copy(x_hbm.at[cur_src.at[j]], buf, sem).wait()
                    pltpu.sync_copy(buf, acc.at[cur_dst.at[j]], add=True)

                    @pl.when(i < n_pairs - 1)
                    def _():
                        pltpu.async_copy(x_hbm.at[cur_src.at[j + 2]], buf, sem)

                    if not last:
                        @pl.when(i == n_pairs - 1)
                        def _():
                            pltpu.async_copy(x_hbm.at[nxt_src.at[b]], buf, sem)

                return carry

            lax.fori_loop(0, n_pairs, pair_body, 0)

        plsc.subcore_barrier()

        def readout_body(j, carry):
            blk = s + j * NS

            @pl.when(blk < nblk)
            def _():
                pltpu.async_copy(acc.at[pl.ds(blk * RBLK, RBLK)],
                                 out_hbm.at[c, pl.ds(blk * RBLK, RBLK)], sem0)

            return carry

        lax.fori_loop(0, blk_iters, readout_body, 0)

        def readout_drain_body(j, carry):
            blk = s + j * NS

            @pl.when(blk < nblk)
            def _():
                pltpu.make_async_copy(acc.at[pl.ds(blk * RBLK, RBLK)],
                                      out_hbm.at[c, pl.ds(blk * RBLK, RBLK)],
                                      sem0).wait()

            return carry

        lax.fori_loop(0, blk_iters, readout_drain_body, 0)

    return seg_sum(x, edges3d)


def _tc_body(p_ref, wgc_ref, bgc_ref, wlow_ref, blow_ref, h_ref, lg_ref):
    p = p_ref[...]
    agg = p[0] + p[1]
    h = jnp.dot(agg, wgc_ref[...], preferred_element_type=jnp.float32) + bgc_ref[...]
    h_ref[...] = h
    t = jnp.dot(h, wlow_ref[...], preferred_element_type=jnp.float32) + blow_ref[...]
    nrm = jnp.sqrt(jnp.sum(t * t, axis=1, keepdims=True))
    lg_ref[...] = t / jnp.maximum(nrm, 1e-12)


def kernel(x, edge_index, W_gc, b_gc, W_low, b_low):
    n, nfeat = x.shape
    out = W_gc.shape[1]
    clus = W_low.shape[1]
    e = edge_index.shape[1]

    edges3d = edge_index.reshape(2, e // CHUNK, CHUNK)

    partials = _sc_segment_sum(x, edges3d)

    bn = 2000
    grid = n // bn
    h, logits = pl.pallas_call(
        _tc_body,
        grid=(grid,),
        in_specs=[
            pl.BlockSpec((NC, bn, nfeat), lambda i: (0, i, 0)),
            pl.BlockSpec((nfeat, out), lambda i: (0, 0)),
            pl.BlockSpec((1, out), lambda i: (0, 0)),
            pl.BlockSpec((out, clus), lambda i: (0, 0)),
            pl.BlockSpec((1, clus), lambda i: (0, 0)),
        ],
        out_specs=[
            pl.BlockSpec((bn, out), lambda i: (i, 0)),
            pl.BlockSpec((bn, clus), lambda i: (i, 0)),
        ],
        out_shape=[
            jax.ShapeDtypeStruct((n, out), jnp.float32),
            jax.ShapeDtypeStruct((n, clus), jnp.float32),
        ],
    )(partials, W_gc, b_gc.reshape(1, out), W_low, b_low.reshape(1, clus))
    return (h, logits)


# speedup vs baseline: 1.0102x; 1.0102x over previous
import functools

import jax
import jax.numpy as jnp
from jax import lax
from jax.experimental import pallas as pl
from jax.experimental.pallas import tpu as pltpu
from jax.experimental.pallas import tpu_sc as plsc

NC = 2
NS = 16
NW = NC * NS
CHUNK = 125
RBLK = 80


def _sc_segment_sum(x, edges3d):
    n, f = x.shape
    _, chunks_total, chunk = edges3d.shape
    chunks_w = chunks_total // NW
    nblk = n // RBLK
    blk_iters = (nblk + NS - 1) // NS
    phases = 2
    chunks_ph = chunks_w // phases
    mesh = plsc.VectorSubcoreMesh(core_axis_name="c", subcore_axis_name="s")

    @functools.partial(
        pl.kernel,
        out_type=jax.ShapeDtypeStruct((NC, n, f), jnp.float32),
        mesh=mesh,
        scratch_types=[
            pltpu.VMEM((chunks_ph, chunk), jnp.int32),
            pltpu.VMEM((chunks_ph, chunk), jnp.int32),
            pltpu.VMEM((chunk, f), jnp.float32),
            pltpu.VMEM((chunk, f), jnp.float32),
            pltpu.VMEM_SHARED((n, f), jnp.float32),
            pltpu.SemaphoreType.DMA,
            pltpu.SemaphoreType.DMA,
            pltpu.SemaphoreType.DMA,
        ],
    )
    def seg_sum(x_hbm, e_hbm, out_hbm, src_v, dst_v, rows0, rows1,
                acc, sem0, sem1, zsem):
        c = lax.axis_index("c")
        s = lax.axis_index("s")
        wid = c * NS + s

        zeros16 = jnp.zeros((16,), jnp.float32)

        def zero_body(i, carry):
            r = i // (f // 16)
            l = i - r * (f // 16)
            rows0[r, pl.ds(l * 16, 16)] = zeros16
            return carry

        lax.fori_loop(0, RBLK * (f // 16), zero_body, 0)

        def zero_acc_body(j, carry):
            blk = s + j * NS

            @pl.when(blk < nblk)
            def _():
                pltpu.async_copy(rows0.at[pl.ds(0, RBLK)],
                                 acc.at[pl.ds(blk * RBLK, RBLK)], zsem)

            return carry

        lax.fori_loop(0, blk_iters, zero_acc_body, 0)

        pltpu.sync_copy(e_hbm.at[0, pl.ds(wid * chunks_w, chunks_ph)], src_v)
        pltpu.sync_copy(e_hbm.at[1, pl.ds(wid * chunks_w, chunks_ph)], dst_v)
        pltpu.async_copy(x_hbm.at[src_v.at[0]], rows1, sem1)

        def zero_drain_body(j, carry):
            blk = s + j * NS

            @pl.when(blk < nblk)
            def _():
                pltpu.make_async_copy(rows0.at[pl.ds(0, RBLK)],
                                      acc.at[pl.ds(blk * RBLK, RBLK)], zsem).wait()

            return carry

        lax.fori_loop(0, blk_iters, zero_drain_body, 0)

        plsc.subcore_barrier()

        n_pairs = chunks_ph // 2
        for phase in range(phases):
            if phase == 0:
                bufs = ((rows1, sem1), (rows0, sem0))
            else:
                bufs = ((rows0, sem0), (rows1, sem1))
                base = wid * chunks_w + phase * chunks_ph
                pltpu.sync_copy(e_hbm.at[0, pl.ds(base, chunks_ph)], src_v)
                pltpu.sync_copy(e_hbm.at[1, pl.ds(base, chunks_ph)], dst_v)
                pltpu.async_copy(x_hbm.at[src_v.at[0]], bufs[0][0], bufs[0][1])
            pltpu.async_copy(x_hbm.at[src_v.at[1]], bufs[1][0], bufs[1][1])

            def pair_body(i, carry, bufs=bufs):
                for b, (buf, sem) in enumerate(bufs):
                    j = 2 * i + b
                    pltpu.make_async_copy(x_hbm.at[src_v.at[j]], buf, sem).wait()
                    pltpu.sync_copy(buf, acc.at[dst_v.at[j]], add=True)

                    @pl.when(i < n_pairs - 1)
                    def _():
                        pltpu.async_copy(x_hbm.at[src_v.at[j + 2]], buf, sem)

                return carry

            lax.fori_loop(0, n_pairs, pair_body, 0)

        plsc.subcore_barrier()

        def readout_body(j, carry):
            blk = s + j * NS

            @pl.when(blk < nblk)
            def _():
                pltpu.async_copy(acc.at[pl.ds(blk * RBLK, RBLK)],
                                 out_hbm.at[c, pl.ds(blk * RBLK, RBLK)], sem0)

            return carry

        lax.fori_loop(0, blk_iters, readout_body, 0)

        def readout_drain_body(j, carry):
            blk = s + j * NS

            @pl.when(blk < nblk)
            def _():
                pltpu.make_async_copy(acc.at[pl.ds(blk * RBLK, RBLK)],
                                      out_hbm.at[c, pl.ds(blk * RBLK, RBLK)],
                                      sem0).wait()

            return carry

        lax.fori_loop(0, blk_iters, readout_drain_body, 0)

    return seg_sum(x, edges3d)


def _tc_body(p_ref, wgc_ref, bgc_ref, wlow_ref, blow_ref, h_ref, lg_ref):
    p = p_ref[...]
    agg = p[0] + p[1]
    h = jnp.dot(agg, wgc_ref[...], preferred_element_type=jnp.float32) + bgc_ref[...]
    h_ref[...] = h
    t = jnp.dot(h, wlow_ref[...], preferred_element_type=jnp.float32) + blow_ref[...]
    nrm = jnp.sqrt(jnp.sum(t * t, axis=1, keepdims=True))
    lg_ref[...] = t / jnp.maximum(nrm, 1e-12)


def kernel(x, edge_index, W_gc, b_gc, W_low, b_low):
    n, nfeat = x.shape
    out = W_gc.shape[1]
    clus = W_low.shape[1]
    e = edge_index.shape[1]

    edges3d = edge_index.reshape(2, e // CHUNK, CHUNK)

    partials = _sc_segment_sum(x, edges3d)

    bn = 2000
    grid = n // bn
    h, logits = pl.pallas_call(
        _tc_body,
        grid=(grid,),
        in_specs=[
            pl.BlockSpec((NC, bn, nfeat), lambda i: (0, i, 0)),
            pl.BlockSpec((nfeat, out), lambda i: (0, 0)),
            pl.BlockSpec((1, out), lambda i: (0, 0)),
            pl.BlockSpec((out, clus), lambda i: (0, 0)),
            pl.BlockSpec((1, clus), lambda i: (0, 0)),
        ],
        out_specs=[
            pl.BlockSpec((bn, out), lambda i: (i, 0)),
            pl.BlockSpec((bn, clus), lambda i: (i, 0)),
        ],
        out_shape=[
            jax.ShapeDtypeStruct((n, out), jnp.float32),
            jax.ShapeDtypeStruct((n, clus), jnp.float32),
        ],
    )(partials, W_gc, b_gc.reshape(1, out), W_low, b_low.reshape(1, clus))
    return (h, logits)


# speedup vs baseline: 1.0271x; 1.0167x over previous
import functools

import jax
import jax.numpy as jnp
from jax import lax
from jax.experimental import pallas as pl
from jax.experimental.pallas import tpu as pltpu
from jax.experimental.pallas import tpu_sc as plsc

NC = 2
NS = 16
NW = NC * NS
CHUNK = 125
RBLK = 80


def _sc_segment_sum(x, edges3d):
    n, f = x.shape
    _, chunks_total, chunk = edges3d.shape
    chunks_w = chunks_total // NW
    nblk = n // RBLK
    blk_iters = (nblk + NS - 1) // NS
    phases = 2
    chunks_ph = chunks_w // phases
    mesh = plsc.VectorSubcoreMesh(core_axis_name="c", subcore_axis_name="s")

    @functools.partial(
        pl.kernel,
        out_type=jax.ShapeDtypeStruct((NC, n, f), jnp.float32),
        mesh=mesh,
        scratch_types=[
            pltpu.VMEM((chunks_ph, chunk), jnp.int32),
            pltpu.VMEM((chunks_ph, chunk), jnp.int32),
            pltpu.VMEM((chunk, f), jnp.float32),
            pltpu.VMEM((chunk, f), jnp.float32),
            pltpu.VMEM_SHARED((n, f), jnp.float32),
            pltpu.SemaphoreType.DMA,
            pltpu.SemaphoreType.DMA,
            pltpu.SemaphoreType.DMA,
        ],
    )
    def seg_sum(x_hbm, e_hbm, out_hbm, src_v, dst_v, rows0, rows1,
                acc, sem0, sem1, zsem):
        c = lax.axis_index("c")
        s = lax.axis_index("s")
        wid = c * NS + s

        zeros16 = jnp.zeros((16,), jnp.float32)

        def zero_body(i, carry):
            r = i // (f // 16)
            l = i - r * (f // 16)
            rows0[r, pl.ds(l * 16, 16)] = zeros16
            return carry

        lax.fori_loop(0, RBLK * (f // 16), zero_body, 0)

        def zero_acc_body(j, carry):
            blk = s + j * NS

            @pl.when(blk < nblk)
            def _():
                pltpu.async_copy(rows0.at[pl.ds(0, RBLK)],
                                 acc.at[pl.ds(blk * RBLK, RBLK)], zsem)

            return carry

        lax.fori_loop(0, blk_iters, zero_acc_body, 0)

        pltpu.sync_copy(e_hbm.at[0, pl.ds(wid * chunks_w, chunks_ph)], src_v)
        pltpu.sync_copy(e_hbm.at[1, pl.ds(wid * chunks_w, chunks_ph)], dst_v)
        pltpu.async_copy(x_hbm.at[src_v.at[0]], rows1, sem1)

        def zero_drain_body(j, carry):
            blk = s + j * NS

            @pl.when(blk < nblk)
            def _():
                pltpu.make_async_copy(rows0.at[pl.ds(0, RBLK)],
                                      acc.at[pl.ds(blk * RBLK, RBLK)], zsem).wait()

            return carry

        lax.fori_loop(0, blk_iters, zero_drain_body, 0)

        plsc.subcore_barrier()

        n_pairs = chunks_ph // 2
        for phase in range(phases):
            if phase == 0:
                bufs = ((rows1, sem1), (rows0, sem0))
            else:
                bufs = ((rows0, sem0), (rows1, sem1))
                base = wid * chunks_w + phase * chunks_ph
                pltpu.sync_copy(e_hbm.at[0, pl.ds(base, chunks_ph)], src_v)
                pltpu.sync_copy(e_hbm.at[1, pl.ds(base, chunks_ph)], dst_v)
                pltpu.async_copy(x_hbm.at[src_v.at[0]], bufs[0][0], bufs[0][1])
            pltpu.async_copy(x_hbm.at[src_v.at[1]], bufs[1][0], bufs[1][1])

            def pair_body(i, carry, bufs=bufs):
                for b, (buf, sem) in enumerate(bufs):
                    j = 2 * i + b
                    pltpu.make_async_copy(x_hbm.at[src_v.at[j]], buf, sem).wait()
                    pltpu.sync_copy(buf, acc.at[dst_v.at[j]], add=True)

                    @pl.when(i < n_pairs - 1)
                    def _():
                        pltpu.async_copy(x_hbm.at[src_v.at[j + 2]], buf, sem)

                return carry

            lax.fori_loop(0, n_pairs, pair_body, 0)

        plsc.subcore_barrier()

        def readout_body(j, carry):
            blk = s + j * NS

            @pl.when(blk < nblk)
            def _():
                pltpu.async_copy(acc.at[pl.ds(blk * RBLK, RBLK)],
                                 out_hbm.at[c, pl.ds(blk * RBLK, RBLK)], sem0)

            return carry

        lax.fori_loop(0, blk_iters, readout_body, 0)

        def readout_drain_body(j, carry):
            blk = s + j * NS

            @pl.when(blk < nblk)
            def _():
                pltpu.make_async_copy(acc.at[pl.ds(blk * RBLK, RBLK)],
                                      out_hbm.at[c, pl.ds(blk * RBLK, RBLK)],
                                      sem0).wait()

            return carry

        lax.fori_loop(0, blk_iters, readout_drain_body, 0)

    return seg_sum(x, edges3d)


def _tc_body(p_ref, wgc_ref, bgc_ref, wlow_ref, blow_ref, h_ref, lg_ref):
    p = p_ref[...]
    agg = p[0] + p[1]
    h = jnp.dot(agg, wgc_ref[...], preferred_element_type=jnp.float32) + bgc_ref[...]
    h_ref[...] = h
    t = jnp.dot(h, wlow_ref[...], preferred_element_type=jnp.float32) + blow_ref[...]
    nrm = jnp.sqrt(jnp.sum(t * t, axis=1, keepdims=True))
    lg_ref[...] = t / jnp.maximum(nrm, 1e-12)


def kernel(x, edge_index, W_gc, b_gc, W_low, b_low):
    n, nfeat = x.shape
    out = W_gc.shape[1]
    clus = W_low.shape[1]
    e = edge_index.shape[1]

    edges3d = edge_index.reshape(2, e // CHUNK, CHUNK)

    partials = _sc_segment_sum(x, edges3d)

    bn = 5000
    grid = n // bn
    h, logits = pl.pallas_call(
        _tc_body,
        grid=(grid,),
        in_specs=[
            pl.BlockSpec((NC, bn, nfeat), lambda i: (0, i, 0)),
            pl.BlockSpec((nfeat, out), lambda i: (0, 0)),
            pl.BlockSpec((1, out), lambda i: (0, 0)),
            pl.BlockSpec((out, clus), lambda i: (0, 0)),
            pl.BlockSpec((1, clus), lambda i: (0, 0)),
        ],
        out_specs=[
            pl.BlockSpec((bn, out), lambda i: (i, 0)),
            pl.BlockSpec((bn, clus), lambda i: (i, 0)),
        ],
        out_shape=[
            jax.ShapeDtypeStruct((n, out), jnp.float32),
            jax.ShapeDtypeStruct((n, clus), jnp.float32),
        ],
    )(partials, W_gc, b_gc.reshape(1, out), W_low, b_low.reshape(1, clus))
    return (h, logits)
